# baseline (speedup 1.0000x reference)
import functools

import jax
import jax.numpy as jnp
from jax import lax
from jax.experimental import pallas as pl
from jax.experimental.pallas import tpu as pltpu
from jax.experimental.pallas import tpu_sc as plsc

N_U = 25000
N_I = 25000
EDGES = 800000
D = 64
H = 32
NP = 32768
PT = 2048
K = 128
RING = 12
L1 = 6
NB = 34
EPT = NB * RING * K
EP = 16 * EPT
DUMMY = 25000


def _pass_kernel(scale):
    mesh = plsc.VectorSubcoreMesh(core_axis_name="c", subcore_axis_name="s")
    f32 = jnp.float32

    @functools.partial(
        pl.kernel,
        out_type=(
            jax.ShapeDtypeStruct((NP, H), f32),
            jax.ShapeDtypeStruct((NP, H), f32),
            jax.ShapeDtypeStruct((NP, H), f32),
            jax.ShapeDtypeStruct((NP, H), f32),
        ),
        mesh=mesh,
        scratch_types=[
            pltpu.VMEM_SHARED((NP, H), f32),
            pltpu.VMEM((RING, K), jnp.int32),
            pltpu.VMEM((RING, K), jnp.int32),
            pltpu.VMEM((RING, K, H), f32),
            pltpu.VMEM((K, 16), f32),
            pltpu.SemaphoreType.DMA,
        ],
        compiler_params=pltpu.CompilerParams(use_tc_tiling_on_sc=False),
    )
    def kfn(tab0, tab1, bidx, norm, accin0, accin1, zeros_h,
            out0, out1, accout0, accout1,
            acc_sp, gv, sv, gbufs, nv, gsem):
        c = lax.axis_index("c")
        s = lax.axis_index("s")
        r0 = s * PT

        pltpu.sync_copy(zeros_h, gbufs.at[1])

        def zero_body(ch, carry):
            pltpu.sync_copy(gbufs.at[1], acc_sp.at[pl.ds(r0 + ch * K, K), :])
            return carry

        lax.fori_loop(0, PT // K, zero_body, 0)
        plsc.subcore_barrier()

        def edge_phase(tab):
            def blk_body(b, carry):
                pltpu.sync_copy(bidx.at[s, b, 0], gv)
                pltpu.sync_copy(bidx.at[s, b, 1], sv)
                gds = [
                    pltpu.async_copy(tab.at[gv.at[j]], gbufs.at[j], gsem)
                    for j in range(RING)
                ]
                for j in range(RING):
                    gds[j].wait()
                    pltpu.sync_copy(gbufs.at[j], acc_sp.at[sv.at[j]],
                                    add=True)
                return carry

            lax.fori_loop(0, NB, blk_body, 0)

        @pl.when(c == 0)
        def _():
            edge_phase(tab0)

        @pl.when(c == 1)
        def _():
            edge_phase(tab1)

        plsc.subcore_barrier()

        def writeout(accin, out, accout):
            def ch_body(ch, carry):
                base = r0 + ch * K
                pltpu.sync_copy(acc_sp.at[pl.ds(base, K), :], gbufs.at[0])
                pltpu.sync_copy(accin.at[pl.ds(base, K), :], gbufs.at[2])
                pltpu.sync_copy(norm.at[pl.ds(base, K), :], nv)

                def row_body(i, carry2):
                    nsplat = nv[i, pl.ds(0, 16)]
                    for c2 in range(H // 16):
                        a = gbufs[0, i, pl.ds(c2 * 16, 16)]
                        o = a * nsplat
                        gbufs[1, i, pl.ds(c2 * 16, 16)] = o
                        ao = (gbufs[2, i, pl.ds(c2 * 16, 16)] + o) * scale
                        gbufs[3, i, pl.ds(c2 * 16, 16)] = ao
                    return carry2

                lax.fori_loop(0, K, row_body, 0)
                pltpu.sync_copy(gbufs.at[1], out.at[pl.ds(base, K), :])
                pltpu.sync_copy(gbufs.at[3], accout.at[pl.ds(base, K), :])
                return carry

            lax.fori_loop(0, PT // K, ch_body, 0)

        @pl.when(c == 0)
        def _():
            writeout(accin0, out0, accout0)

        @pl.when(c == 1)
        def _():
            writeout(accin1, out1, accout1)

    return kfn


def _deg_kernel():
    mesh = plsc.VectorSubcoreMesh(core_axis_name="c", subcore_axis_name="s")
    f32 = jnp.float32
    W = 16

    @functools.partial(
        pl.kernel,
        out_type=(
            jax.ShapeDtypeStruct((NP, W), f32),
            jax.ShapeDtypeStruct((NP, W), f32),
        ),
        mesh=mesh,
        scratch_types=[
            pltpu.VMEM_SHARED((NP, W), f32),
            pltpu.VMEM((RING, K), jnp.int32),
            pltpu.VMEM((K, W), f32),
            pltpu.VMEM((K, W), f32),
            pltpu.VMEM((K, W), f32),
            pltpu.SemaphoreType.DMA,
        ],
        compiler_params=pltpu.CompilerParams(use_tc_tiling_on_sc=False),
    )
    def kfn(s1, s2, ones_h, zeros_w,
            norm_i, norm_u,
            dacc, sv, ones_v, dbuf, nbuf, ssem):
        c = lax.axis_index("c")
        s = lax.axis_index("s")
        r0 = s * PT

        pltpu.sync_copy(zeros_w, dbuf)

        def zero_body(ch, carry):
            pltpu.sync_copy(dbuf, dacc.at[pl.ds(r0 + ch * K, K), :])
            return carry

        lax.fori_loop(0, PT // K, zero_body, 0)
        pltpu.sync_copy(ones_h, ones_v)
        plsc.subcore_barrier()

        def count_phase(bidx):
            def blk_body(b, carry):
                pltpu.sync_copy(bidx.at[s, b, 1], sv)
                descs = [
                    pltpu.async_copy(ones_v, dacc.at[sv.at[j]], ssem, add=True)
                    for j in range(RING)
                ]
                for d in descs:
                    d.wait()
                return carry

            lax.fori_loop(0, NB, blk_body, 0)

        @pl.when(c == 0)
        def _():
            count_phase(s1)

        @pl.when(c == 1)
        def _():
            count_phase(s2)

        plsc.subcore_barrier()

        def writeout(nout):
            def ch_body(ch, carry):
                base = r0 + ch * K
                pltpu.sync_copy(dacc.at[pl.ds(base, K), :], dbuf)

                def row_body(i, carry2):
                    deg = dbuf[i, pl.ds(0, W)]
                    nbuf[i, pl.ds(0, W)] = 1.0 / jnp.maximum(deg, 1.0)
                    return carry2

                lax.fori_loop(0, K, row_body, 0)
                pltpu.sync_copy(nbuf, nout.at[pl.ds(base, K), :])
                return carry

            lax.fori_loop(0, PT // K, ch_body, 0)

        @pl.when(c == 0)
        def _():
            writeout(norm_i)

        @pl.when(c == 1)
        def _():
            writeout(norm_u)

    return kfn


def _pad_idx(x, fill):
    x = jnp.concatenate([x, jnp.full((EP - EDGES,), fill, jnp.int32)])
    return x.reshape(16, NB, 1, RING, K)


def kernel(user_emb, item_emb, edge_index):
    f32 = jnp.float32
    src = edge_index[0]
    dst = edge_index[1]

    b1 = jnp.concatenate(
        [_pad_idx(src, 0), _pad_idx(dst, DUMMY)], axis=2)
    b2 = jnp.concatenate(
        [_pad_idx(dst, 0), _pad_idx(src, DUMMY)], axis=2)

    pad_rows = ((0, NP - N_U), (0, 0))
    u0 = jnp.pad(user_emb[:, :H], pad_rows)
    u1 = jnp.pad(user_emb[:, H:], pad_rows)
    i0 = jnp.pad(item_emb[:, :H], pad_rows)
    i1 = jnp.pad(item_emb[:, H:], pad_rows)

    zeros_h = jnp.zeros((K, H), f32)
    ones_w = jnp.ones((K, 16), f32)
    zeros_w = jnp.zeros((K, 16), f32)

    norm_i, norm_u = _deg_kernel()(b1, b2, ones_w, zeros_w)

    p_mid = _pass_kernel(1.0)
    p_fin = _pass_kernel(1.0 / 3.0)

    rst0, rst1, iacc0, iacc1 = p_mid(u0, u1, b1, norm_i, i0, i1, zeros_h)
    bs0, bs1, uacc0, uacc1 = p_mid(rst0, rst1, b2, norm_u, u0, u1, zeros_h)
    rst0, rst1, iacc0, iacc1 = p_fin(bs0, bs1, b1, norm_i, iacc0, iacc1, zeros_h)
    _, _, uacc0, uacc1 = p_fin(rst0, rst1, b2, norm_u, uacc0, uacc1, zeros_h)

    user_out = jnp.concatenate([uacc0[:N_U], uacc1[:N_U]], axis=1)
    item_out = jnp.concatenate([iacc0[:N_I], iacc1[:N_I]], axis=1)
    return jnp.concatenate([user_out, item_out], axis=0)


# speedup vs baseline: 2.0722x; 2.0722x over previous
import functools

import jax
import jax.numpy as jnp
from jax import lax
from jax.experimental import pallas as pl
from jax.experimental.pallas import tpu as pltpu
from jax.experimental.pallas import tpu_sc as plsc

N_U = 25000
N_I = 25000
EDGES = 800000
D = 64
H = 32
NP = 32768
PT = 2048
K = 128
RING = 8
L1 = 6
NB = 49
EPT = NB * RING * K
EP = 16 * EPT
DUMMY = 25000


def _pass_kernel(scale):
    mesh = plsc.VectorSubcoreMesh(core_axis_name="c", subcore_axis_name="s")
    f32 = jnp.float32

    @functools.partial(
        pl.kernel,
        out_type=(
            jax.ShapeDtypeStruct((NP, H), f32),
            jax.ShapeDtypeStruct((NP, H), f32),
            jax.ShapeDtypeStruct((NP, H), f32),
            jax.ShapeDtypeStruct((NP, H), f32),
        ),
        mesh=mesh,
        scratch_types=[
            pltpu.VMEM_SHARED((NP, H), f32),
            pltpu.VMEM((RING, K), jnp.int32),
            pltpu.VMEM((RING, K), jnp.int32),
            pltpu.VMEM((RING, K, H), f32),
            pltpu.VMEM((K, 16), f32),
            pltpu.SemaphoreType.DMA,
        ],
        compiler_params=pltpu.CompilerParams(use_tc_tiling_on_sc=False),
    )
    def kfn(tab0, tab1, bidx, norm, accin0, accin1, zeros_h,
            out0, out1, accout0, accout1,
            acc_sp, gv, sv, gbufs, nv, gsem):
        c = lax.axis_index("c")
        s = lax.axis_index("s")
        r0 = s * PT

        pltpu.sync_copy(zeros_h, gbufs.at[1])

        def zero_body(ch, carry):
            pltpu.sync_copy(gbufs.at[1], acc_sp.at[pl.ds(r0 + ch * K, K), :])
            return carry

        lax.fori_loop(0, PT // K, zero_body, 0)
        plsc.subcore_barrier()

        def edge_phase(tab):
            def blk_body(b, carry):
                pltpu.sync_copy(bidx.at[s, b, 0], gv)
                pltpu.sync_copy(bidx.at[s, b, 1], sv)
                gds = [
                    pltpu.async_copy(tab.at[gv.at[j]], gbufs.at[j], gsem)
                    for j in range(RING)
                ]
                for j in range(RING):
                    gds[j].wait()
                    pltpu.sync_copy(gbufs.at[j], acc_sp.at[sv.at[j]],
                                    add=True)
                return carry

            lax.fori_loop(0, NB, blk_body, 0)

        @pl.when(c == 0)
        def _():
            edge_phase(tab0)

        @pl.when(c == 1)
        def _():
            edge_phase(tab1)

        plsc.subcore_barrier()

        def writeout(accin, out, accout):
            def ch_body(ch, carry):
                base = r0 + ch * K
                pltpu.sync_copy(acc_sp.at[pl.ds(base, K), :], gbufs.at[0])
                pltpu.sync_copy(accin.at[pl.ds(base, K), :], gbufs.at[2])
                pltpu.sync_copy(norm.at[pl.ds(base, K), :], nv)

                def row_body(i, carry2):
                    nsplat = nv[i, pl.ds(0, 16)]
                    for c2 in range(H // 16):
                        a = gbufs[0, i, pl.ds(c2 * 16, 16)]
                        o = a * nsplat
                        gbufs[1, i, pl.ds(c2 * 16, 16)] = o
                        ao = (gbufs[2, i, pl.ds(c2 * 16, 16)] + o) * scale
                        gbufs[3, i, pl.ds(c2 * 16, 16)] = ao
                    return carry2

                lax.fori_loop(0, K, row_body, 0)
                pltpu.sync_copy(gbufs.at[1], out.at[pl.ds(base, K), :])
                pltpu.sync_copy(gbufs.at[3], accout.at[pl.ds(base, K), :])
                return carry

            lax.fori_loop(0, PT // K, ch_body, 0)

        @pl.when(c == 0)
        def _():
            writeout(accin0, out0, accout0)

        @pl.when(c == 1)
        def _():
            writeout(accin1, out1, accout1)

    return kfn


def _deg_kernel():
    mesh = plsc.VectorSubcoreMesh(core_axis_name="c", subcore_axis_name="s")
    f32 = jnp.float32
    W = 16

    @functools.partial(
        pl.kernel,
        out_type=(
            jax.ShapeDtypeStruct((NP, W), f32),
            jax.ShapeDtypeStruct((NP, W), f32),
        ),
        mesh=mesh,
        scratch_types=[
            pltpu.VMEM_SHARED((NP, W), f32),
            pltpu.VMEM((RING, K), jnp.int32),
            pltpu.VMEM((K, W), f32),
            pltpu.VMEM((K, W), f32),
            pltpu.VMEM((K, W), f32),
            pltpu.SemaphoreType.DMA,
        ],
        compiler_params=pltpu.CompilerParams(use_tc_tiling_on_sc=False),
    )
    def kfn(s1, s2, ones_h, zeros_w,
            norm_i, norm_u,
            dacc, sv, ones_v, dbuf, nbuf, ssem):
        c = lax.axis_index("c")
        s = lax.axis_index("s")
        r0 = s * PT

        pltpu.sync_copy(zeros_w, dbuf)

        def zero_body(ch, carry):
            pltpu.sync_copy(dbuf, dacc.at[pl.ds(r0 + ch * K, K), :])
            return carry

        lax.fori_loop(0, PT // K, zero_body, 0)
        pltpu.sync_copy(ones_h, ones_v)
        plsc.subcore_barrier()

        def count_phase(bidx):
            def blk_body(b, carry):
                pltpu.sync_copy(bidx.at[s, b, 1], sv)
                descs = [
                    pltpu.async_copy(ones_v, dacc.at[sv.at[j]], ssem, add=True)
                    for j in range(RING)
                ]
                for d in descs:
                    d.wait()
                return carry

            lax.fori_loop(0, NB, blk_body, 0)

        @pl.when(c == 0)
        def _():
            count_phase(s1)

        @pl.when(c == 1)
        def _():
            count_phase(s2)

        plsc.subcore_barrier()

        def writeout(nout):
            def ch_body(ch, carry):
                base = r0 + ch * K
                pltpu.sync_copy(dacc.at[pl.ds(base, K), :], dbuf)

                def row_body(i, carry2):
                    deg = dbuf[i, pl.ds(0, W)]
                    nbuf[i, pl.ds(0, W)] = 1.0 / jnp.maximum(deg, 1.0)
                    return carry2

                lax.fori_loop(0, K, row_body, 0)
                pltpu.sync_copy(nbuf, nout.at[pl.ds(base, K), :])
                return carry

            lax.fori_loop(0, PT // K, ch_body, 0)

        @pl.when(c == 0)
        def _():
            writeout(norm_i)

        @pl.when(c == 1)
        def _():
            writeout(norm_u)

    return kfn


def _pad_idx(x, fill):
    x = jnp.concatenate([x, jnp.full((EP - EDGES,), fill, jnp.int32)])
    return x.reshape(16, NB, 1, RING, K)


def kernel(user_emb, item_emb, edge_index):
    f32 = jnp.float32
    src = edge_index[0]
    dst = edge_index[1]

    b1 = jnp.concatenate(
        [_pad_idx(src, 0), _pad_idx(dst, DUMMY)], axis=2)
    b2 = jnp.concatenate(
        [_pad_idx(dst, 0), _pad_idx(src, DUMMY)], axis=2)

    pad_rows = ((0, NP - N_U), (0, 0))
    u0 = jnp.pad(user_emb[:, :H], pad_rows)
    u1 = jnp.pad(user_emb[:, H:], pad_rows)
    i0 = jnp.pad(item_emb[:, :H], pad_rows)
    i1 = jnp.pad(item_emb[:, H:], pad_rows)

    zeros_h = jnp.zeros((K, H), f32)
    ones_w = jnp.ones((K, 16), f32)
    zeros_w = jnp.zeros((K, 16), f32)

    norm_i, norm_u = _deg_kernel()(b1, b2, ones_w, zeros_w)

    p_mid = _pass_kernel(1.0)
    p_fin = _pass_kernel(1.0 / 3.0)

    rst0, rst1, iacc0, iacc1 = p_mid(u0, u1, b1, norm_i, i0, i1, zeros_h)
    bs0, bs1, uacc0, uacc1 = p_mid(rst0, rst1, b2, norm_u, u0, u1, zeros_h)
    rst0, rst1, iacc0, iacc1 = p_fin(bs0, bs1, b1, norm_i, iacc0, iacc1, zeros_h)
    _, _, uacc0, uacc1 = p_fin(rst0, rst1, b2, norm_u, uacc0, uacc1, zeros_h)

    user_out = jnp.concatenate([uacc0[:N_U], uacc1[:N_U]], axis=1)
    item_out = jnp.concatenate([iacc0[:N_I], iacc1[:N_I]], axis=1)
    return jnp.concatenate([user_out, item_out], axis=0)


# speedup vs baseline: 2.2830x; 1.1017x over previous
import functools

import jax
import jax.numpy as jnp
from jax import lax
from jax.experimental import pallas as pl
from jax.experimental.pallas import tpu as pltpu
from jax.experimental.pallas import tpu_sc as plsc

N_U = 25000
N_I = 25000
EDGES = 800000
D = 64
H = 32
NP = 32768
PT = 2048
K = 128
RING = 8
L1 = 6
NB = 49
EPT = NB * RING * K
EP = 16 * EPT
DUMMY = 25000


def _pass_kernel(scale):
    mesh = plsc.VectorSubcoreMesh(core_axis_name="c", subcore_axis_name="s")
    f32 = jnp.float32

    @functools.partial(
        pl.kernel,
        out_type=(
            jax.ShapeDtypeStruct((NP, H), f32),
            jax.ShapeDtypeStruct((NP, H), f32),
            jax.ShapeDtypeStruct((NP, H), f32),
            jax.ShapeDtypeStruct((NP, H), f32),
        ),
        mesh=mesh,
        scratch_types=[
            pltpu.VMEM_SHARED((NP, H), f32),
            pltpu.VMEM((2 * RING, K), jnp.int32),
            pltpu.VMEM((RING, K, H), f32),
            pltpu.VMEM((K, 16), f32),
            pltpu.SemaphoreType.DMA,
            pltpu.SemaphoreType.DMA,
        ],
        compiler_params=pltpu.CompilerParams(use_tc_tiling_on_sc=False),
    )
    def kfn(tab0, tab1, bidx, norm, accin0, accin1, zeros_h,
            out0, out1, accout0, accout1,
            acc_sp, bv, gbufs, nv, gsem, ssem):
        c = lax.axis_index("c")
        s = lax.axis_index("s")
        r0 = s * PT

        pltpu.sync_copy(zeros_h, gbufs.at[1])

        def zero_body(ch, carry):
            pltpu.sync_copy(gbufs.at[1], acc_sp.at[pl.ds(r0 + ch * K, K), :])
            return carry

        lax.fori_loop(0, PT // K, zero_body, 0)
        plsc.subcore_barrier()

        def edge_phase(tab):
            def blk_body(b, carry):
                pltpu.sync_copy(bidx.at[s, b], bv)
                gds = [
                    pltpu.async_copy(tab.at[bv.at[j]], gbufs.at[j], gsem)
                    for j in range(RING)
                ]
                sds = []
                for j in range(RING):
                    gds[j].wait()
                    sds.append(pltpu.async_copy(
                        gbufs.at[j], acc_sp.at[bv.at[RING + j]],
                        ssem, add=True))
                    if j > 0:
                        sds[j - 1].wait()
                sds[RING - 1].wait()
                return carry

            lax.fori_loop(0, NB, blk_body, 0)

        @pl.when(c == 0)
        def _():
            edge_phase(tab0)

        @pl.when(c == 1)
        def _():
            edge_phase(tab1)

        plsc.subcore_barrier()

        def writeout(accin, out, accout):
            def ch_body(ch, carry):
                base = r0 + ch * K
                pltpu.sync_copy(acc_sp.at[pl.ds(base, K), :], gbufs.at[0])
                pltpu.sync_copy(accin.at[pl.ds(base, K), :], gbufs.at[2])
                pltpu.sync_copy(norm.at[pl.ds(base, K), :], nv)

                def row_body(i, carry2):
                    nsplat = nv[i, pl.ds(0, 16)]
                    for c2 in range(H // 16):
                        a = gbufs[0, i, pl.ds(c2 * 16, 16)]
                        o = a * nsplat
                        gbufs[1, i, pl.ds(c2 * 16, 16)] = o
                        ao = (gbufs[2, i, pl.ds(c2 * 16, 16)] + o) * scale
                        gbufs[3, i, pl.ds(c2 * 16, 16)] = ao
                    return carry2

                lax.fori_loop(0, K, row_body, 0)
                pltpu.sync_copy(gbufs.at[1], out.at[pl.ds(base, K), :])
                pltpu.sync_copy(gbufs.at[3], accout.at[pl.ds(base, K), :])
                return carry

            lax.fori_loop(0, PT // K, ch_body, 0)

        @pl.when(c == 0)
        def _():
            writeout(accin0, out0, accout0)

        @pl.when(c == 1)
        def _():
            writeout(accin1, out1, accout1)

    return kfn


def _deg_kernel():
    mesh = plsc.VectorSubcoreMesh(core_axis_name="c", subcore_axis_name="s")
    f32 = jnp.float32
    W = 16

    @functools.partial(
        pl.kernel,
        out_type=(
            jax.ShapeDtypeStruct((NP, W), f32),
            jax.ShapeDtypeStruct((NP, W), f32),
        ),
        mesh=mesh,
        scratch_types=[
            pltpu.VMEM_SHARED((NP, W), f32),
            pltpu.VMEM((RING, K), jnp.int32),
            pltpu.VMEM((K, W), f32),
            pltpu.VMEM((K, W), f32),
            pltpu.VMEM((K, W), f32),
            pltpu.SemaphoreType.DMA,
        ],
        compiler_params=pltpu.CompilerParams(use_tc_tiling_on_sc=False),
    )
    def kfn(s1, s2, ones_h, zeros_w,
            norm_i, norm_u,
            dacc, sv, ones_v, dbuf, nbuf, ssem):
        c = lax.axis_index("c")
        s = lax.axis_index("s")
        r0 = s * PT

        pltpu.sync_copy(zeros_w, dbuf)

        def zero_body(ch, carry):
            pltpu.sync_copy(dbuf, dacc.at[pl.ds(r0 + ch * K, K), :])
            return carry

        lax.fori_loop(0, PT // K, zero_body, 0)
        pltpu.sync_copy(ones_h, ones_v)
        plsc.subcore_barrier()

        def count_phase(bidx):
            def blk_body(b, carry):
                pltpu.sync_copy(bidx.at[s, b, pl.ds(RING, RING)], sv)
                descs = [
                    pltpu.async_copy(ones_v, dacc.at[sv.at[j]], ssem, add=True)
                    for j in range(RING)
                ]
                for d in descs:
                    d.wait()
                return carry

            lax.fori_loop(0, NB, blk_body, 0)

        @pl.when(c == 0)
        def _():
            count_phase(s1)

        @pl.when(c == 1)
        def _():
            count_phase(s2)

        plsc.subcore_barrier()

        def writeout(nout):
            def ch_body(ch, carry):
                base = r0 + ch * K
                pltpu.sync_copy(dacc.at[pl.ds(base, K), :], dbuf)

                def row_body(i, carry2):
                    deg = dbuf[i, pl.ds(0, W)]
                    nbuf[i, pl.ds(0, W)] = 1.0 / jnp.maximum(deg, 1.0)
                    return carry2

                lax.fori_loop(0, K, row_body, 0)
                pltpu.sync_copy(nbuf, nout.at[pl.ds(base, K), :])
                return carry

            lax.fori_loop(0, PT // K, ch_body, 0)

        @pl.when(c == 0)
        def _():
            writeout(norm_i)

        @pl.when(c == 1)
        def _():
            writeout(norm_u)

    return kfn


def _pad_idx(x, fill):
    x = jnp.concatenate([x, jnp.full((EP - EDGES,), fill, jnp.int32)])
    return x.reshape(16, NB, 1, RING, K)


def kernel(user_emb, item_emb, edge_index):
    f32 = jnp.float32
    src = edge_index[0]
    dst = edge_index[1]

    b1 = jnp.concatenate(
        [_pad_idx(src, 0), _pad_idx(dst, DUMMY)],
        axis=2).reshape(16, NB, 2 * RING, K)
    b2 = jnp.concatenate(
        [_pad_idx(dst, 0), _pad_idx(src, DUMMY)],
        axis=2).reshape(16, NB, 2 * RING, K)

    pad_rows = ((0, NP - N_U), (0, 0))
    u0 = jnp.pad(user_emb[:, :H], pad_rows)
    u1 = jnp.pad(user_emb[:, H:], pad_rows)
    i0 = jnp.pad(item_emb[:, :H], pad_rows)
    i1 = jnp.pad(item_emb[:, H:], pad_rows)

    zeros_h = jnp.zeros((K, H), f32)
    ones_w = jnp.ones((K, 16), f32)
    zeros_w = jnp.zeros((K, 16), f32)

    norm_i, norm_u = _deg_kernel()(b1, b2, ones_w, zeros_w)

    p_mid = _pass_kernel(1.0)
    p_fin = _pass_kernel(1.0 / 3.0)

    rst0, rst1, iacc0, iacc1 = p_mid(u0, u1, b1, norm_i, i0, i1, zeros_h)
    bs0, bs1, uacc0, uacc1 = p_mid(rst0, rst1, b2, norm_u, u0, u1, zeros_h)
    rst0, rst1, iacc0, iacc1 = p_fin(bs0, bs1, b1, norm_i, iacc0, iacc1, zeros_h)
    _, _, uacc0, uacc1 = p_fin(rst0, rst1, b2, norm_u, uacc0, uacc1, zeros_h)

    user_out = jnp.concatenate([uacc0[:N_U], uacc1[:N_U]], axis=1)
    item_out = jnp.concatenate([iacc0[:N_I], iacc1[:N_I]], axis=1)
    return jnp.concatenate([user_out, item_out], axis=0)
